# baseline (device time: 60140 ns/iter reference)
import jax
import jax.numpy as jnp
from jax import lax
from jax.experimental import pallas as pl
from jax.experimental.pallas import tpu as pltpu

N_DEV = 4
N_SUB = 2


def kernel(x, router_W, route_idx, expert_W):
    n, d = x.shape
    e_loc, _, h = expert_W.shape
    n_exp = router_W.shape[1]
    chunk = n // N_DEV
    sub = chunk // N_SUB
    n_flows = (N_DEV - 1) * N_SUB

    expert_Wb = expert_W.astype(jnp.bfloat16)

    def body(x_ref, rw_ref, idx_ref, ewb_ref, out_ref,
             part_ref, recv_ref, send_sems, recv_sems):
        my = lax.axis_index("i")

        barrier = pltpu.get_barrier_semaphore()
        for k in range(1, N_DEV):
            peer = lax.rem(my + k, N_DEV)
            pl.semaphore_signal(barrier, inc=1, device_id=(peer,),
                                device_id_type=pl.DeviceIdType.MESH)
        pl.semaphore_wait(barrier, N_DEV - 1)

        lids = my * e_loc + lax.broadcasted_iota(jnp.int32, (1, e_loc), 1)
        iota = lax.broadcasted_iota(jnp.int32, (sub, n_exp), 1)

        def compute_sub(off):
            xc = x_ref[pl.ds(off, sub), :]
            scores = jnp.dot(xc, rw_ref[:, :],
                             preferred_element_type=jnp.float32)
            probs = jax.nn.softmax(scores, axis=-1)
            e0c = idx_ref[pl.ds(off, sub), 0:1]
            e1c = idx_ref[pl.ds(off, sub), 1:2]
            p0c = jnp.sum(jnp.where(iota == e0c, probs, 0.0),
                          axis=1, keepdims=True)
            p1c = jnp.sum(jnp.where(iota == e1c, probs, 0.0),
                          axis=1, keepdims=True)
            gates = (jnp.where(e0c == lids, p0c, 0.0)
                     + jnp.where(e1c == lids, p1c, 0.0)) / (p0c + p1c)
            acc = jnp.dot((xc * gates[:, 0:1]).astype(jnp.bfloat16),
                          ewb_ref[0], preferred_element_type=jnp.float32)
            for j in range(1, e_loc):
                acc = acc + jnp.dot((xc * gates[:, j:j + 1]).astype(jnp.bfloat16),
                                    ewb_ref[j],
                                    preferred_element_type=jnp.float32)
            part_ref[pl.ds(off, sub), :] = acc.astype(jnp.bfloat16)

        rdmas = []
        for k in range(1, N_DEV):
            c = lax.rem(my + k, N_DEV)
            for s in range(N_SUB):
                off = c * chunk + s * sub
                compute_sub(off)
                slot = (k - 1) * N_SUB + s
                rdma = pltpu.make_async_remote_copy(
                    src_ref=part_ref.at[pl.ds(off, sub)],
                    dst_ref=recv_ref.at[slot],
                    send_sem=send_sems.at[slot],
                    recv_sem=recv_sems.at[slot],
                    device_id=(c,), device_id_type=pl.DeviceIdType.MESH,
                )
                rdma.start()
                rdmas.append(rdma)

        for s in range(N_SUB):
            compute_sub(my * chunk + s * sub)

        for rdma in rdmas:
            rdma.wait_recv()
        for s in range(N_SUB):
            out_ref[pl.ds(s * sub, sub), :] = (
                part_ref[pl.ds(my * chunk + s * sub, sub), :].astype(jnp.float32)
                + recv_ref[0 * N_SUB + s, :, :].astype(jnp.float32)
                + recv_ref[1 * N_SUB + s, :, :].astype(jnp.float32)
                + recv_ref[2 * N_SUB + s, :, :].astype(jnp.float32)
            )

        for rdma in rdmas:
            rdma.wait_send()

    return pl.pallas_call(
        body,
        out_shape=jax.ShapeDtypeStruct((chunk, h), jnp.float32),
        in_specs=[pl.BlockSpec(memory_space=pltpu.VMEM)] * 4,
        out_specs=pl.BlockSpec(memory_space=pltpu.VMEM),
        scratch_shapes=[
            pltpu.VMEM((n, h), jnp.bfloat16),
            pltpu.VMEM((n_flows, sub, h), jnp.bfloat16),
            pltpu.SemaphoreType.DMA((n_flows,)),
            pltpu.SemaphoreType.DMA((n_flows,)),
        ],
        compiler_params=pltpu.CompilerParams(
            collective_id=0,
            vmem_limit_bytes=100 * 1024 * 1024,
        ),
    )(x, router_W, route_idx, expert_Wb)


# device time: 53024 ns/iter; 1.1342x vs baseline; 1.1342x over previous
import jax
import jax.numpy as jnp
from jax import lax
from jax.experimental import pallas as pl
from jax.experimental.pallas import tpu as pltpu

N_DEV = 4
N_SUB = 2


def kernel(x, router_W, route_idx, expert_W):
    n, d = x.shape
    e_loc, _, h = expert_W.shape
    n_exp = router_W.shape[1]
    chunk = n // N_DEV
    sub = chunk // N_SUB
    n_flows = (N_DEV - 1) * N_SUB

    def body(x_ref, rw_ref, idx_ref, ew_ref, out_ref,
             part_ref, recv_ref, ewb_ref, send_sems, recv_sems):
        my = lax.axis_index("i")

        barrier = pltpu.get_barrier_semaphore()
        for k in range(1, N_DEV):
            peer = lax.rem(my + k, N_DEV)
            pl.semaphore_signal(barrier, inc=1, device_id=(peer,),
                                device_id_type=pl.DeviceIdType.MESH)
        pl.semaphore_wait(barrier, N_DEV - 1)

        lids = my * e_loc + lax.broadcasted_iota(jnp.int32, (1, e_loc), 1)
        iota = lax.broadcasted_iota(jnp.int32, (sub, n_exp), 1)

        ewb_ref[:, :, :] = ew_ref[:, :, :].astype(jnp.bfloat16)

        def compute_sub(off):
            xc = x_ref[pl.ds(off, sub), :]
            scores = jnp.dot(xc, rw_ref[:, :],
                             preferred_element_type=jnp.float32)
            probs = jax.nn.softmax(scores, axis=-1)
            e0c = idx_ref[pl.ds(off, sub), 0:1]
            e1c = idx_ref[pl.ds(off, sub), 1:2]
            p0c = jnp.sum(jnp.where(iota == e0c, probs, 0.0),
                          axis=1, keepdims=True)
            p1c = jnp.sum(jnp.where(iota == e1c, probs, 0.0),
                          axis=1, keepdims=True)
            gates = (jnp.where(e0c == lids, p0c, 0.0)
                     + jnp.where(e1c == lids, p1c, 0.0)) / (p0c + p1c)
            acc = jnp.dot((xc * gates[:, 0:1]).astype(jnp.bfloat16),
                          ewb_ref[0], preferred_element_type=jnp.float32)
            for j in range(1, e_loc):
                acc = acc + jnp.dot((xc * gates[:, j:j + 1]).astype(jnp.bfloat16),
                                    ewb_ref[j],
                                    preferred_element_type=jnp.float32)
            part_ref[pl.ds(off, sub), :] = acc.astype(jnp.bfloat16)

        rdmas = []
        for k in range(1, N_DEV):
            c = lax.rem(my + k, N_DEV)
            for s in range(N_SUB):
                off = c * chunk + s * sub
                compute_sub(off)
                slot = (k - 1) * N_SUB + s
                rdma = pltpu.make_async_remote_copy(
                    src_ref=part_ref.at[pl.ds(off, sub)],
                    dst_ref=recv_ref.at[slot],
                    send_sem=send_sems.at[slot],
                    recv_sem=recv_sems.at[slot],
                    device_id=(c,), device_id_type=pl.DeviceIdType.MESH,
                )
                rdma.start()
                rdmas.append(rdma)

        for s in range(N_SUB):
            compute_sub(my * chunk + s * sub)

        for rdma in rdmas:
            rdma.wait_recv()
        for s in range(N_SUB):
            out_ref[pl.ds(s * sub, sub), :] = (
                part_ref[pl.ds(my * chunk + s * sub, sub), :].astype(jnp.float32)
                + recv_ref[0 * N_SUB + s, :, :].astype(jnp.float32)
                + recv_ref[1 * N_SUB + s, :, :].astype(jnp.float32)
                + recv_ref[2 * N_SUB + s, :, :].astype(jnp.float32)
            )

        for rdma in rdmas:
            rdma.wait_send()

    return pl.pallas_call(
        body,
        out_shape=jax.ShapeDtypeStruct((chunk, h), jnp.float32),
        in_specs=[pl.BlockSpec(memory_space=pltpu.VMEM)] * 4,
        out_specs=pl.BlockSpec(memory_space=pltpu.VMEM),
        scratch_shapes=[
            pltpu.VMEM((n, h), jnp.bfloat16),
            pltpu.VMEM((n_flows, sub, h), jnp.bfloat16),
            pltpu.VMEM((e_loc, d, h), jnp.bfloat16),
            pltpu.SemaphoreType.DMA((n_flows,)),
            pltpu.SemaphoreType.DMA((n_flows,)),
        ],
        compiler_params=pltpu.CompilerParams(
            collective_id=0,
            vmem_limit_bytes=100 * 1024 * 1024,
        ),
    )(x, router_W, route_idx, expert_W)


# device time: 47901 ns/iter; 1.2555x vs baseline; 1.1069x over previous
import jax
import jax.numpy as jnp
from jax import lax
from jax.experimental import pallas as pl
from jax.experimental.pallas import tpu as pltpu

N_DEV = 4
N_SUB = 2


def kernel(x, router_W, route_idx, expert_W):
    n, d = x.shape
    e_loc, _, h = expert_W.shape
    n_exp = router_W.shape[1]
    chunk = n // N_DEV
    sub = chunk // N_SUB
    n_flows = (N_DEV - 1) * N_SUB

    def body(x_ref, rw_ref, idx_ref, ew_ref, out_ref,
             own_ref, qsend_ref, ssend_ref, qrecv_ref, srecv_ref, ewb_ref,
             qsend_sems, qrecv_sems, ssend_sems, srecv_sems):
        my = lax.axis_index("i")

        barrier = pltpu.get_barrier_semaphore()
        for k in range(1, N_DEV):
            peer = lax.rem(my + k, N_DEV)
            pl.semaphore_signal(barrier, inc=1, device_id=(peer,),
                                device_id_type=pl.DeviceIdType.MESH)
        pl.semaphore_wait(barrier, N_DEV - 1)

        lids = my * e_loc + lax.broadcasted_iota(jnp.int32, (1, e_loc), 1)
        iota = lax.broadcasted_iota(jnp.int32, (sub, n_exp), 1)

        ewb_ref[:, :, :] = ew_ref[:, :, :].astype(jnp.bfloat16)

        def compute_sub(off):
            xc = x_ref[pl.ds(off, sub), :]
            scores = jnp.dot(xc, rw_ref[:, :],
                             preferred_element_type=jnp.float32)
            probs = jax.nn.softmax(scores, axis=-1)
            e0c = idx_ref[pl.ds(off, sub), 0:1]
            e1c = idx_ref[pl.ds(off, sub), 1:2]
            p0c = jnp.sum(jnp.where(iota == e0c, probs, 0.0),
                          axis=1, keepdims=True)
            p1c = jnp.sum(jnp.where(iota == e1c, probs, 0.0),
                          axis=1, keepdims=True)
            gates = (jnp.where(e0c == lids, p0c, 0.0)
                     + jnp.where(e1c == lids, p1c, 0.0)) / (p0c + p1c)
            acc = jnp.dot((xc * gates[:, 0:1]).astype(jnp.bfloat16),
                          ewb_ref[0], preferred_element_type=jnp.float32)
            for j in range(1, e_loc):
                acc = acc + jnp.dot((xc * gates[:, j:j + 1]).astype(jnp.bfloat16),
                                    ewb_ref[j],
                                    preferred_element_type=jnp.float32)
            return acc

        rdmas = []
        for k in range(1, N_DEV):
            c = lax.rem(my + k, N_DEV)
            for s in range(N_SUB):
                off = c * chunk + s * sub
                slot = (k - 1) * N_SUB + s
                acc = compute_sub(off)
                m = jnp.maximum(jnp.max(jnp.abs(acc), axis=1, keepdims=True),
                                1e-20)
                qsend_ref[slot, :, :] = jnp.round(
                    acc * (127.0 / m)).astype(jnp.int8)
                ssend_ref[slot, :, :] = m * (1.0 / 127.0)
                q_rdma = pltpu.make_async_remote_copy(
                    src_ref=qsend_ref.at[slot],
                    dst_ref=qrecv_ref.at[slot],
                    send_sem=qsend_sems.at[slot],
                    recv_sem=qrecv_sems.at[slot],
                    device_id=(c,), device_id_type=pl.DeviceIdType.MESH,
                )
                s_rdma = pltpu.make_async_remote_copy(
                    src_ref=ssend_ref.at[slot],
                    dst_ref=srecv_ref.at[slot],
                    send_sem=ssend_sems.at[slot],
                    recv_sem=srecv_sems.at[slot],
                    device_id=(c,), device_id_type=pl.DeviceIdType.MESH,
                )
                q_rdma.start()
                s_rdma.start()
                rdmas.append(q_rdma)
                rdmas.append(s_rdma)

        for s in range(N_SUB):
            own_ref[pl.ds(s * sub, sub), :] = compute_sub(my * chunk + s * sub)

        for rdma in rdmas:
            rdma.wait_recv()
        for s in range(N_SUB):
            total = own_ref[pl.ds(s * sub, sub), :]
            for k in range(N_DEV - 1):
                slot = k * N_SUB + s
                total = total + (qrecv_ref[slot, :, :].astype(jnp.float32)
                                 * srecv_ref[slot, :, :])
            out_ref[pl.ds(s * sub, sub), :] = total

        for rdma in rdmas:
            rdma.wait_send()

    return pl.pallas_call(
        body,
        out_shape=jax.ShapeDtypeStruct((chunk, h), jnp.float32),
        in_specs=[pl.BlockSpec(memory_space=pltpu.VMEM)] * 4,
        out_specs=pl.BlockSpec(memory_space=pltpu.VMEM),
        scratch_shapes=[
            pltpu.VMEM((chunk, h), jnp.float32),
            pltpu.VMEM((n_flows, sub, h), jnp.int8),
            pltpu.VMEM((n_flows, sub, 1), jnp.float32),
            pltpu.VMEM((n_flows, sub, h), jnp.int8),
            pltpu.VMEM((n_flows, sub, 1), jnp.float32),
            pltpu.VMEM((e_loc, d, h), jnp.bfloat16),
            pltpu.SemaphoreType.DMA((n_flows,)),
            pltpu.SemaphoreType.DMA((n_flows,)),
            pltpu.SemaphoreType.DMA((n_flows,)),
            pltpu.SemaphoreType.DMA((n_flows,)),
        ],
        compiler_params=pltpu.CompilerParams(
            collective_id=0,
            vmem_limit_bytes=100 * 1024 * 1024,
        ),
    )(x, router_W, route_idx, expert_W)
